# baseline (speedup 1.0000x reference)
import jax
import jax.numpy as jnp
from jax.experimental import pallas as pl
from jax.experimental.pallas import tpu as pltpu

_NSPLIT = 4


def _dma_body(z_ref, o1_ref, o2_ref, *sems):
    n = z_ref.shape[0]
    cols = z_ref.shape[1] // 2
    w = cols // _NSPLIT
    copies = []
    for j in range(_NSPLIT):
        copies.append(pltpu.make_async_copy(
            z_ref.at[:, pl.ds(j * w, w)],
            o1_ref.at[:, pl.ds(j * w, w)],
            sems[j]))
        copies.append(pltpu.make_async_copy(
            z_ref.at[:, pl.ds(cols + j * w, w)],
            o2_ref.at[:, pl.ds(j * w, w)],
            sems[_NSPLIT + j]))
    for c in copies:
        c.start()
    for c in copies:
        c.wait()


def kernel(z):
    n, c, h, w = z.shape
    ch = c // 2
    cols = ch * h * w
    z2d = z.reshape(n, 2 * cols)

    out1, out2 = pl.pallas_call(
        _dma_body,
        in_specs=[pl.BlockSpec(memory_space=pl.ANY)],
        out_specs=[
            pl.BlockSpec(memory_space=pl.ANY),
            pl.BlockSpec(memory_space=pl.ANY),
        ],
        out_shape=[
            jax.ShapeDtypeStruct((n, cols), z.dtype),
            jax.ShapeDtypeStruct((n, cols), z.dtype),
        ],
        scratch_shapes=[pltpu.SemaphoreType.DMA] * (2 * _NSPLIT),
    )(z2d)

    z1 = out1.reshape(n, ch, h, w)
    z2 = out2.reshape(n, ch, h, w)
    log_det = jnp.zeros((), z.dtype)
    return (z1, z2, log_det)


# speedup vs baseline: 7.6925x; 7.6925x over previous
import jax
import jax.numpy as jnp
from jax.experimental import pallas as pl


def _split_body(z1_ref, z2_ref, a_ref, b_ref):
    a_ref[...] = z1_ref[...]
    b_ref[...] = z2_ref[...]


def kernel(z):
    n, c, h, w = z.shape
    ch = c // 2

    z1, z2 = pl.pallas_call(
        _split_body,
        grid=(n,),
        in_specs=[
            pl.BlockSpec((1, ch, h, w), lambda i: (i, 0, 0, 0)),
            pl.BlockSpec((1, ch, h, w), lambda i: (i, 1, 0, 0)),
        ],
        out_specs=[
            pl.BlockSpec((1, ch, h, w), lambda i: (i, 0, 0, 0)),
            pl.BlockSpec((1, ch, h, w), lambda i: (i, 0, 0, 0)),
        ],
        out_shape=[
            jax.ShapeDtypeStruct((n, ch, h, w), z.dtype),
            jax.ShapeDtypeStruct((n, ch, h, w), z.dtype),
        ],
    )(z, z)

    log_det = jnp.zeros((), z.dtype)
    return (z1, z2, log_det)
